# baseline (speedup 1.0000x reference)
import jax
import jax.numpy as jnp
from jax.experimental import pallas as pl
from jax.experimental.pallas import tpu as pltpu


def _copy_kernel(emb_ref, out_ref, sem):
    copy = pltpu.make_async_copy(emb_ref, out_ref, sem)
    copy.start()
    copy.wait()


def kernel(edge_index, emb):
    del edge_index
    return pl.pallas_call(
        _copy_kernel,
        in_specs=[pl.BlockSpec(memory_space=pl.ANY)],
        out_specs=pl.BlockSpec(memory_space=pl.ANY),
        scratch_shapes=[pltpu.SemaphoreType.DMA],
        out_shape=jax.ShapeDtypeStruct(emb.shape, emb.dtype),
    )(emb)


# speedup vs baseline: 1.0004x; 1.0004x over previous
import jax
import jax.numpy as jnp
from jax.experimental import pallas as pl
from jax.experimental.pallas import tpu as pltpu


_N_CHUNKS = 16


def _copy_kernel(emb_ref, out_ref, *sems):
    rows = emb_ref.shape[0] // _N_CHUNKS
    copies = [
        pltpu.make_async_copy(
            emb_ref.at[pl.ds(i * rows, rows), :],
            out_ref.at[pl.ds(i * rows, rows), :],
            sems[i],
        )
        for i in range(_N_CHUNKS)
    ]
    for c in copies:
        c.start()
    for c in copies:
        c.wait()


def kernel(edge_index, emb):
    del edge_index
    return pl.pallas_call(
        _copy_kernel,
        in_specs=[pl.BlockSpec(memory_space=pl.ANY)],
        out_specs=pl.BlockSpec(memory_space=pl.ANY),
        scratch_shapes=[pltpu.SemaphoreType.DMA] * _N_CHUNKS,
        out_shape=jax.ShapeDtypeStruct(emb.shape, emb.dtype),
    )(emb)


# speedup vs baseline: 42.3295x; 42.3115x over previous
import jax
import jax.numpy as jnp
from jax.experimental import pallas as pl
from jax.experimental.pallas import tpu as pltpu


_BLOCK_ROWS = 4000


def _copy_kernel(emb_ref, out_ref):
    out_ref[...] = emb_ref[...]


def kernel(edge_index, emb):
    del edge_index
    n, c = emb.shape
    return pl.pallas_call(
        _copy_kernel,
        grid=(n // _BLOCK_ROWS,),
        in_specs=[pl.BlockSpec((_BLOCK_ROWS, c), lambda i: (i, 0))],
        out_specs=pl.BlockSpec((_BLOCK_ROWS, c), lambda i: (i, 0)),
        out_shape=jax.ShapeDtypeStruct(emb.shape, emb.dtype),
    )(emb)
